# baseline (speedup 1.0000x reference)
import functools

import jax
import jax.numpy as jnp
from jax import lax
from jax.experimental import pallas as pl
from jax.experimental.pallas import tpu as pltpu
from jax.experimental.pallas import tpu_sc as plsc

_BLK = 2000
_HBLK = _BLK // 2
_CHUNK = 80


def _rne_hi16(v):
    return v + (0x7FFF + (lax.shift_right_logical(v, 16) & 1))


def _pack_words(x):
    h = x.shape[1] // 2
    a = lax.bitcast_convert_type(x[:, :h], jnp.int32)
    b = lax.bitcast_convert_type(x[:, h:], jnp.int32)
    return lax.shift_right_logical(_rne_hi16(a), 16) | (
        _rne_hi16(b) & jnp.int32(-65536))


def _proj_body(nf_ref, ws_ref, wd_ref, p_ref, q_ref):
    nf = nf_ref[...]
    p_ref[...] = _pack_words(
        jnp.dot(nf, ws_ref[...], preferred_element_type=jnp.float32))
    q_ref[...] = _pack_words(
        jnp.dot(nf, wd_ref[...], preferred_element_type=jnp.float32))


def _node_proj(node_feat, w_s, w_d):
    n, d = node_feat.shape
    out_dim = w_s.shape[1]
    blk = 2000
    grid = n // blk
    return pl.pallas_call(
        _proj_body,
        grid=(grid,),
        in_specs=[
            pl.BlockSpec((blk, d), lambda i: (i, 0)),
            pl.BlockSpec((d, out_dim), lambda i: (0, 0)),
            pl.BlockSpec((d, out_dim), lambda i: (0, 0)),
        ],
        out_specs=[
            pl.BlockSpec((blk, out_dim // 2), lambda i: (i, 0)),
            pl.BlockSpec((blk, out_dim // 2), lambda i: (i, 0)),
        ],
        out_shape=[
            jax.ShapeDtypeStruct((n, out_dim // 2), jnp.int32),
            jax.ShapeDtypeStruct((n, out_dim // 2), jnp.int32),
        ],
        compiler_params=pltpu.CompilerParams(
            dimension_semantics=("parallel",),
        ),
    )(node_feat, w_s, w_d)


def _sc_gather_add(p, q, src, dst):
    n, dw = p.shape
    d = dw * 2
    e = src.shape[0]
    info = plsc.get_sparse_core_info()
    nc, ns = info.num_cores, info.num_subcores
    nw = nc * ns
    per_w = e // nw
    chunk = _CHUNK
    hp = chunk // 2
    n_chunks = per_w // chunk
    half = (n_chunks - 1) // 2
    cpb = _HBLK // hp
    mesh = plsc.VectorSubcoreMesh(core_axis_name="c", subcore_axis_name="s")

    @functools.partial(
        pl.kernel,
        mesh=mesh,
        compiler_params=pltpu.CompilerParams(
            needs_layout_passes=False, use_tc_tiling_on_sc=False),
        out_type=jax.ShapeDtypeStruct((e // 2, d), jnp.int32),
        scratch_types=[
            pltpu.VMEM((per_w,), jnp.int32),
            pltpu.VMEM((per_w,), jnp.int32),
            pltpu.VMEM((chunk, dw), jnp.int32),
            pltpu.VMEM((chunk, dw), jnp.int32),
            pltpu.VMEM((chunk, dw), jnp.int32),
            pltpu.VMEM((chunk, dw), jnp.int32),
            pltpu.VMEM((hp, d), jnp.int32),
            pltpu.VMEM((hp, d), jnp.int32),
            pltpu.SemaphoreType.DMA,
            pltpu.SemaphoreType.DMA,
            pltpu.SemaphoreType.DMA,
            pltpu.SemaphoreType.DMA,
        ],
    )
    def k(p_hbm, q_hbm, src_hbm, dst_hbm, out_hbm,
          idxs, idxd, bufp0, bufq0, bufp1, bufq1, outb0, outb1,
          gsem0, gsem1, osem0, osem1):
        wid = lax.axis_index("s") * nc + lax.axis_index("c")
        w_base = pl.multiple_of(wid * per_w, 8)
        w_base2 = pl.multiple_of(wid * (per_w // 2), 8)
        bufp = (bufp0, bufp1)
        bufq = (bufq0, bufq1)
        outb = (outb0, outb1)
        gsem = (gsem0, gsem1)
        osem = (osem0, osem1)

        pltpu.sync_copy(src_hbm.at[pl.ds(w_base, per_w)], idxs)
        pltpu.sync_copy(dst_hbm.at[pl.ds(w_base, per_w)], idxd)

        def lo_off(c):
            fb = c // cpb
            t = c % cpb
            return pl.multiple_of(fb * _BLK + t * hp, 8)

        def fire_gathers(c, s):
            off = pl.multiple_of(lo_off(c) + (c % cpb) * hp, 8)
            pltpu.async_copy(
                p_hbm.at[idxs.at[pl.ds(off, chunk)]], bufp[s], gsem[s])
            pltpu.async_copy(
                q_hbm.at[idxd.at[pl.ds(off, chunk)]], bufq[s], gsem[s])

        def wait_gathers(s):
            pltpu.make_async_copy(
                p_hbm.at[idxs.at[pl.ds(0, chunk)]], bufp[s], gsem[s]).wait()
            pltpu.make_async_copy(
                q_hbm.at[idxd.at[pl.ds(0, chunk)]], bufq[s], gsem[s]).wait()

        def add_pack(s):
            bp, bq, ob = bufp[s], bufq[s], outb[s]

            def bfadd(x, y):
                return plsc.bitcast(
                    plsc.bitcast(x, jnp.bfloat16) + plsc.bitcast(y, jnp.bfloat16),
                    jnp.int32)

            def pair_body(j, cc):
                for g in range(dw // 16):
                    sl = pl.ds(g * 16, 16)
                    sa = bfadd(bp[j, sl], bq[j, sl])
                    sb = bfadd(bp[j + hp, sl], bq[j + hp, sl])
                    ob[j, sl] = (sa & 0xFFFF) | lax.shift_left(sb, 16)
                    ob[j, pl.ds(dw + g * 16, 16)] = (
                        lax.shift_right_logical(sa, 16) | (sb & jnp.int32(-65536)))
                return cc

            lax.fori_loop(0, hp, pair_body, 0)

        def fire_out(c, s):
            fb = c // cpb
            t = c % cpb
            row = pl.multiple_of(w_base2 + fb * _HBLK + t * hp, 8)
            pltpu.async_copy(outb[s], out_hbm.at[pl.ds(row, hp)], osem[s])

        def wait_out(s):
            pltpu.make_async_copy(
                outb[s], out_hbm.at[pl.ds(w_base2, hp)], osem[s]).wait()

        fire_gathers(0, 0)

        def body(t, carry):
            c0 = 2 * t
            fire_gathers(c0 + 1, 1)
            wait_gathers(0)

            @pl.when(t > 0)
            def _():
                wait_out(0)

            add_pack(0)
            fire_out(c0, 0)
            fire_gathers(c0 + 2, 0)
            wait_gathers(1)

            @pl.when(t > 0)
            def _():
                wait_out(1)

            add_pack(1)
            fire_out(c0 + 1, 1)
            return carry

        lax.fori_loop(0, half, body, 0)
        wait_gathers(0)
        wait_out(0)
        add_pack(0)
        fire_out(n_chunks - 1, 0)
        wait_out(0)
        wait_out(1)

    return k(p, q, src, dst)


def _final_body(g2_ref, ea_ref, we_ref, b_ref, out_ref):
    w = g2_ref[...]
    lo = lax.bitcast_convert_type(w << 16, jnp.float32)
    hi = lax.bitcast_convert_type(w & jnp.int32(-65536), jnp.float32)
    g = jnp.concatenate([lo, hi], axis=0)
    mm = jnp.dot(ea_ref[...], we_ref[...], preferred_element_type=jnp.float32)
    out_ref[...] = g + mm + b_ref[...]


def _final(g2, edge_attr, w_e, b2d):
    e2, d = g2.shape
    e = e2 * 2
    d_edge = edge_attr.shape[1]
    grid = e // _BLK
    return pl.pallas_call(
        _final_body,
        grid=(grid,),
        in_specs=[
            pl.BlockSpec((_HBLK, d), lambda i: (i, 0)),
            pl.BlockSpec((_BLK, d_edge), lambda i: (i, 0)),
            pl.BlockSpec((d_edge, d), lambda i: (0, 0)),
            pl.BlockSpec((1, d), lambda i: (0, 0)),
        ],
        out_specs=pl.BlockSpec((_BLK, d), lambda i: (i, 0)),
        out_shape=jax.ShapeDtypeStruct((e, d), jnp.float32),
        compiler_params=pltpu.CompilerParams(
            dimension_semantics=("parallel",),
        ),
    )(g2, edge_attr, w_e, b2d)


def kernel(node_feat, edge_attr, edge_index, W, b):
    d_edge = edge_attr.shape[1]
    d_feat = node_feat.shape[1]
    w_e = W[:d_edge]
    w_s = W[d_edge:d_edge + d_feat]
    w_d = W[d_edge + d_feat:]
    e = edge_index.shape[1]
    nblk = e // _BLK
    cpb = _HBLK // (_CHUNK // 2)
    hp = _CHUNK // 2
    idx2 = edge_index.reshape(2, nblk, 2, cpb, hp).transpose(0, 1, 3, 2, 4)
    idx2 = idx2.reshape(2, e)
    src = idx2[0]
    dst = idx2[1]
    p, q = _node_proj(node_feat, w_s, w_d)
    g2 = _sc_gather_add(p, q, src, dst)
    return _final(g2, edge_attr, w_e, b.reshape(1, -1))


# speedup vs baseline: 1.1553x; 1.1553x over previous
import functools

import jax
import jax.numpy as jnp
from jax import lax
from jax.experimental import pallas as pl
from jax.experimental.pallas import tpu as pltpu
from jax.experimental.pallas import tpu_sc as plsc

_BLK = 2000
_HBLK = _BLK // 2
_CHUNK = 80


def _rne_hi16(v):
    return v + (0x7FFF + (lax.shift_right_logical(v, 16) & 1))


def _pack_words(x):
    h = x.shape[1] // 2
    a = lax.bitcast_convert_type(x[:, :h], jnp.int32)
    b = lax.bitcast_convert_type(x[:, h:], jnp.int32)
    return lax.shift_right_logical(_rne_hi16(a), 16) | (
        _rne_hi16(b) & jnp.int32(-65536))


def _proj_body(nf_ref, ws_ref, wd_ref, p_ref, q_ref):
    nf = nf_ref[...]
    p_ref[...] = _pack_words(
        jnp.dot(nf, ws_ref[...], preferred_element_type=jnp.float32))
    q_ref[...] = _pack_words(
        jnp.dot(nf, wd_ref[...], preferred_element_type=jnp.float32))


def _node_proj(node_feat, w_s, w_d):
    n, d = node_feat.shape
    out_dim = w_s.shape[1]
    blk = 2000
    grid = n // blk
    return pl.pallas_call(
        _proj_body,
        grid=(grid,),
        in_specs=[
            pl.BlockSpec((blk, d), lambda i: (i, 0)),
            pl.BlockSpec((d, out_dim), lambda i: (0, 0)),
            pl.BlockSpec((d, out_dim), lambda i: (0, 0)),
        ],
        out_specs=[
            pl.BlockSpec((blk, out_dim // 2), lambda i: (i, 0)),
            pl.BlockSpec((blk, out_dim // 2), lambda i: (i, 0)),
        ],
        out_shape=[
            jax.ShapeDtypeStruct((n, out_dim // 2), jnp.int32),
            jax.ShapeDtypeStruct((n, out_dim // 2), jnp.int32),
        ],
        compiler_params=pltpu.CompilerParams(
            dimension_semantics=("parallel",),
        ),
    )(node_feat, w_s, w_d)


def _sc_gather_add(p, q, src, dst):
    n, dw = p.shape
    d = dw * 2
    e = src.shape[0]
    info = plsc.get_sparse_core_info()
    nc, ns = info.num_cores, info.num_subcores
    nw = nc * ns
    per_w = e // nw
    chunk = _CHUNK
    hp = chunk // 2
    n_chunks = per_w // chunk
    half = (n_chunks - 1) // 2
    cpb = _HBLK // hp
    mesh = plsc.VectorSubcoreMesh(core_axis_name="c", subcore_axis_name="s")

    @functools.partial(
        pl.kernel,
        mesh=mesh,
        compiler_params=pltpu.CompilerParams(
            needs_layout_passes=False, use_tc_tiling_on_sc=False),
        out_type=jax.ShapeDtypeStruct((e // 2, d), jnp.int32),
        scratch_types=[
            pltpu.VMEM((per_w,), jnp.int32),
            pltpu.VMEM((per_w,), jnp.int32),
            pltpu.VMEM((chunk, dw), jnp.int32),
            pltpu.VMEM((chunk, dw), jnp.int32),
            pltpu.VMEM((chunk, dw), jnp.int32),
            pltpu.VMEM((chunk, dw), jnp.int32),
            pltpu.VMEM((hp, d), jnp.int32),
            pltpu.VMEM((hp, d), jnp.int32),
            pltpu.SemaphoreType.DMA,
            pltpu.SemaphoreType.DMA,
            pltpu.SemaphoreType.DMA,
            pltpu.SemaphoreType.DMA,
        ],
    )
    def k(p_hbm, q_hbm, src_hbm, dst_hbm, out_hbm,
          idxs, idxd, bufp0, bufq0, bufp1, bufq1, outb0, outb1,
          gsem0, gsem1, osem0, osem1):
        wid = lax.axis_index("s") * nc + lax.axis_index("c")
        w_base = pl.multiple_of(wid * per_w, 8)
        w_base2 = pl.multiple_of(wid * (per_w // 2), 8)
        bufp = (bufp0, bufp1)
        bufq = (bufq0, bufq1)
        outb = (outb0, outb1)
        gsem = (gsem0, gsem1)
        osem = (osem0, osem1)

        pltpu.sync_copy(src_hbm.at[pl.ds(w_base, per_w)], idxs)
        pltpu.sync_copy(dst_hbm.at[pl.ds(w_base, per_w)], idxd)

        def lo_off(c):
            fb = c // cpb
            t = c % cpb
            return pl.multiple_of(fb * _BLK + t * hp, 8)

        def fire_gathers(c, s):
            off = pl.multiple_of(lo_off(c), 8)
            pltpu.async_copy(
                p_hbm.at[idxs.at[pl.ds(off, hp)]], bufp[s].at[pl.ds(0, hp)], gsem[s])
            pltpu.async_copy(
                q_hbm.at[idxd.at[pl.ds(off, hp)]], bufq[s].at[pl.ds(0, hp)], gsem[s])
            pltpu.async_copy(
                p_hbm.at[idxs.at[pl.ds(pl.multiple_of(off + _HBLK, 8), hp)]],
                bufp[s].at[pl.ds(hp, hp)], gsem[s])
            pltpu.async_copy(
                q_hbm.at[idxd.at[pl.ds(pl.multiple_of(off + _HBLK, 8), hp)]],
                bufq[s].at[pl.ds(hp, hp)], gsem[s])

        def wait_gathers(s):
            pltpu.make_async_copy(
                p_hbm.at[idxs.at[pl.ds(0, hp)]], bufp[s].at[pl.ds(0, hp)], gsem[s]).wait()
            pltpu.make_async_copy(
                q_hbm.at[idxd.at[pl.ds(0, hp)]], bufq[s].at[pl.ds(0, hp)], gsem[s]).wait()
            pltpu.make_async_copy(
                p_hbm.at[idxs.at[pl.ds(0, hp)]], bufp[s].at[pl.ds(hp, hp)], gsem[s]).wait()
            pltpu.make_async_copy(
                q_hbm.at[idxd.at[pl.ds(0, hp)]], bufq[s].at[pl.ds(hp, hp)], gsem[s]).wait()

        def add_pack(s):
            bp, bq, ob = bufp[s], bufq[s], outb[s]

            def bfadd(x, y):
                return plsc.bitcast(
                    plsc.bitcast(x, jnp.bfloat16) + plsc.bitcast(y, jnp.bfloat16),
                    jnp.int32)

            def pair_body(j, cc):
                for g in range(dw // 16):
                    sl = pl.ds(g * 16, 16)
                    sa = bfadd(bp[j, sl], bq[j, sl])
                    sb = bfadd(bp[j + hp, sl], bq[j + hp, sl])
                    ob[j, sl] = (sa & 0xFFFF) | lax.shift_left(sb, 16)
                    ob[j, pl.ds(dw + g * 16, 16)] = (
                        lax.shift_right_logical(sa, 16) | (sb & jnp.int32(-65536)))
                return cc

            lax.fori_loop(0, hp, pair_body, 0)

        def fire_out(c, s):
            fb = c // cpb
            t = c % cpb
            row = pl.multiple_of(w_base2 + fb * _HBLK + t * hp, 8)
            pltpu.async_copy(outb[s], out_hbm.at[pl.ds(row, hp)], osem[s])

        def wait_out(s):
            pltpu.make_async_copy(
                outb[s], out_hbm.at[pl.ds(w_base2, hp)], osem[s]).wait()

        fire_gathers(0, 0)

        def body(t, carry):
            c0 = 2 * t
            fire_gathers(c0 + 1, 1)
            wait_gathers(0)

            @pl.when(t > 0)
            def _():
                wait_out(0)

            add_pack(0)
            fire_out(c0, 0)
            fire_gathers(c0 + 2, 0)
            wait_gathers(1)

            @pl.when(t > 0)
            def _():
                wait_out(1)

            add_pack(1)
            fire_out(c0 + 1, 1)
            return carry

        lax.fori_loop(0, half, body, 0)
        wait_gathers(0)
        wait_out(0)
        add_pack(0)
        fire_out(n_chunks - 1, 0)
        wait_out(0)
        wait_out(1)

    return k(p, q, src, dst)


def _final_body(g2_ref, ea_ref, we_ref, b_ref, out_ref):
    w = g2_ref[...]
    lo = lax.bitcast_convert_type(w << 16, jnp.float32)
    hi = lax.bitcast_convert_type(w & jnp.int32(-65536), jnp.float32)
    g = jnp.concatenate([lo, hi], axis=0)
    mm = jnp.dot(ea_ref[...], we_ref[...], preferred_element_type=jnp.float32)
    out_ref[...] = g + mm + b_ref[...]


def _final(g2, edge_attr, w_e, b2d):
    e2, d = g2.shape
    e = e2 * 2
    d_edge = edge_attr.shape[1]
    grid = e // _BLK
    return pl.pallas_call(
        _final_body,
        grid=(grid,),
        in_specs=[
            pl.BlockSpec((_HBLK, d), lambda i: (i, 0)),
            pl.BlockSpec((_BLK, d_edge), lambda i: (i, 0)),
            pl.BlockSpec((d_edge, d), lambda i: (0, 0)),
            pl.BlockSpec((1, d), lambda i: (0, 0)),
        ],
        out_specs=pl.BlockSpec((_BLK, d), lambda i: (i, 0)),
        out_shape=jax.ShapeDtypeStruct((e, d), jnp.float32),
        compiler_params=pltpu.CompilerParams(
            dimension_semantics=("parallel",),
        ),
    )(g2, edge_attr, w_e, b2d)


def kernel(node_feat, edge_attr, edge_index, W, b):
    d_edge = edge_attr.shape[1]
    d_feat = node_feat.shape[1]
    w_e = W[:d_edge]
    w_s = W[d_edge:d_edge + d_feat]
    w_d = W[d_edge + d_feat:]
    src = edge_index[0]
    dst = edge_index[1]
    p, q = _node_proj(node_feat, w_s, w_d)
    g2 = _sc_gather_add(p, q, src, dst)
    return _final(g2, edge_attr, w_e, b.reshape(1, -1))
